# baseline (speedup 1.0000x reference)
import jax
import jax.numpy as jnp
from jax.experimental import pallas as pl
from jax.experimental.pallas import tpu as pltpu

_K = 1000
_M = 16384
_N = 128
_SLABS = (240, 240, 240, 184, 96)
_NBUF = 2
_OCH = 2048
_NOCH = _M // _OCH


def _mm_kslab(xt_hbm, w_ref, b_ref, o_hbm, xbuf, acc, tbuf, insem, outsem):
    offs = [sum(_SLABS[:i]) for i in range(len(_SLABS))]

    def in_copy(c, slot):
        return pltpu.make_async_copy(
            xt_hbm.at[pl.ds(offs[c], _SLABS[c]), :],
            xbuf.at[slot, pl.ds(0, _SLABS[c])],
            insem.at[slot],
        )

    def out_copy(j, slot):
        return pltpu.make_async_copy(
            tbuf.at[slot], o_hbm.at[pl.ds(j * _OCH, _OCH), :], outsem.at[slot]
        )

    nch = len(_SLABS)
    for c in range(_NBUF - 1):
        in_copy(c, c % _NBUF).start()
    for c in range(nch):
        nxt = c + _NBUF - 1
        if nxt < nch:
            in_copy(nxt, nxt % _NBUF).start()
        in_copy(c, c % _NBUF).wait()
        part = jax.lax.dot_general(
            w_ref[pl.ds(offs[c], _SLABS[c]), :],
            xbuf[c % _NBUF, pl.ds(0, _SLABS[c])],
            (((0,), (0,)), ((), ())),
            preferred_element_type=jnp.float32,
        )
        if c == 0:
            acc[...] = part
        else:
            acc[...] = acc[...] + part
    for j in range(_NOCH):
        slot = j % 2
        if j >= 2:
            out_copy(j - 2, slot).wait()
        y = jnp.maximum(acc[:, pl.ds(j * _OCH, _OCH)] + b_ref[...], 0.0)
        tbuf[slot] = jnp.swapaxes(y, 0, 1)
        out_copy(j, slot).start()
    for j in range(_NOCH - 2, _NOCH):
        out_copy(j, j % 2).wait()


@jax.jit
def _run(inputs, weights, bias_col):
    m, k = inputs.shape
    n = weights.shape[1]
    xt = inputs.T
    return pl.pallas_call(
        _mm_kslab,
        in_specs=[
            pl.BlockSpec(memory_space=pltpu.MemorySpace.HBM),
            pl.BlockSpec(memory_space=pltpu.MemorySpace.VMEM),
            pl.BlockSpec(memory_space=pltpu.MemorySpace.VMEM),
        ],
        out_specs=pl.BlockSpec(memory_space=pltpu.MemorySpace.HBM),
        out_shape=jax.ShapeDtypeStruct((m, n), jnp.float32),
        scratch_shapes=[
            pltpu.VMEM((_NBUF, max(_SLABS), _M), jnp.float32),
            pltpu.VMEM((_N, _M), jnp.float32),
            pltpu.VMEM((2, _OCH, _N), jnp.float32),
            pltpu.SemaphoreType.DMA((_NBUF,)),
            pltpu.SemaphoreType.DMA((2,)),
        ],
    )(xt, weights, bias_col)


def kernel(inputs, kernel, bias):
    return _run(inputs, kernel, bias.reshape(-1, 1))
